# baseline (speedup 1.0000x reference)
import functools

import jax
import jax.numpy as jnp
from jax import lax
from jax.experimental import pallas as pl
from jax.experimental.pallas import tpu as pltpu
from jax.experimental.pallas import tpu_sc as plsc

_N = 207
_F = 2
_P = 12
_B = 12
_MID = _N * 5
_E = 1722
_BP = _B * _P
_C = _BP * _F
_CP = 512
_CH = _CP // 2

_NP = 256
_EP = 2048
_MP = 1152

_ET = _EP // 16

_PREC = jax.lax.Precision.HIGHEST


def _prep_body(col_ref, xnm_ref, wz_ref, wh_ref, lz_ref, lh_ref,
               bz_ref, lzb_ref, bh_ref, lhb_ref,
               xs_ref, dis_ref, cz_ref, ch_ref, dz_ref, dh_ref):
    col = col_ref[...]
    ids = jax.lax.broadcasted_iota(jnp.int32, (_NP, _EP), 0)
    oh_col = (ids == col).astype(jnp.float32)
    deg = jnp.sum(oh_col, axis=1, keepdims=True)
    dis = jnp.where(deg > 0.0, jax.lax.rsqrt(deg), 0.0)
    dis_ref[...] = dis
    xs_ref[0:_NP, :] = xnm_ref[0:_NP, :] * dis
    xs_ref[_NP:, :] = xnm_ref[_NP:, :] * dis
    cz_ref[...] = jax.lax.dot_general(wz_ref[...], lz_ref[...],
                                      (((0,), (1,)), ((), ())),
                                      preferred_element_type=jnp.float32,
                                      precision=_PREC)
    ch_ref[...] = jax.lax.dot_general(wh_ref[...], lh_ref[...],
                                      (((0,), (1,)), ((), ())),
                                      preferred_element_type=jnp.float32,
                                      precision=_PREC)
    dz_ref[...] = jax.lax.dot_general(bz_ref[...], lz_ref[...],
                                      (((1,), (1,)), ((), ())),
                                      preferred_element_type=jnp.float32,
                                      precision=_PREC) + lzb_ref[...]
    dh_ref[...] = jax.lax.dot_general(bh_ref[...], lh_ref[...],
                                      (((1,), (1,)), ((), ())),
                                      preferred_element_type=jnp.float32,
                                      precision=_PREC) + lhb_ref[...]


def _sc_agg_body(row_hbm, col_hbm, xs_hbm, zero_hbm, y_hbm,
                 row_v, col_v, rows_v, acc, sem):
    c = lax.axis_index("c")
    s = lax.axis_index("s")
    base = s * _ET

    pltpu.sync_copy(zero_hbm, acc)
    pltpu.sync_copy(row_hbm.at[pl.ds(base, _ET)], row_v)
    pltpu.sync_copy(col_hbm.at[pl.ds(base, _ET)], col_v)

    def _adj(i, _):
        row_v[pl.ds(i * 16, 16)] = row_v[pl.ds(i * 16, 16)] + c * _NP
        return 0
    lax.fori_loop(0, _ET // 16, _adj, 0)

    pltpu.async_copy(xs_hbm.at[row_v], rows_v, sem).wait()

    lanes = jax.lax.broadcasted_iota(jnp.int32, (16,), 0)

    def _edge_group(g, _):
        for k in range(16):
            eidx = jnp.full((16,), g * 16 + k, jnp.int32)
            ce = plsc.load_gather(col_v, [eidx])
            for j in range(_CH // 16):
                cidx = lanes + (j * 16)
                vals = plsc.load_gather(rows_v, [eidx, cidx])
                plsc.addupdate_scatter(acc, [ce, cidx], vals)
        return 0
    lax.fori_loop(0, _ET // 16, _edge_group, 0)

    pltpu.sync_copy(acc, y_hbm.at[c * 16 + s])


def _reduce_body(yp_ref, out_ref):
    out_ref[0] = jnp.sum(yp_ref[0], axis=0)


def _gate_body(y0_ref, y1_ref, dis_ref, cz_ref, ch_ref,
               dz_ref, dh_ref, att_ref, lin_ref, linb_ref, out_ref, acc_ref):
    p = pl.program_id(1)
    dis = dis_ref[...]
    y0 = y0_ref[0] * dis
    y1 = y1_ref[0] * dis
    pz = y0 * cz_ref[0:1, :] + y1 * cz_ref[1:2, :] + dz_ref[...]
    ph = y0 * ch_ref[0:1, :] + y1 * ch_ref[1:2, :] + dh_ref[...]
    hp = (1.0 - jax.nn.sigmoid(pz)) * jnp.tanh(ph)
    att = att_ref[...]
    e = jnp.exp(att - jnp.max(att))
    w_all = e / jnp.sum(e)
    lane = jax.lax.broadcasted_iota(jnp.int32, (1, _P), 1)
    w = jnp.sum(jnp.where(lane == p, w_all, 0.0))
    contrib = w * hp

    @pl.when(p == 0)
    def _init():
        acc_ref[...] = contrib

    @pl.when(p > 0)
    def _accum():
        acc_ref[...] = acc_ref[...] + contrib

    @pl.when(p == _P - 1)
    def _head():
        h = jnp.maximum(acc_ref[...], 0.0)
        o = jax.lax.dot_general(h, lin_ref[...],
                                (((1,), (1,)), ((), ())),
                                preferred_element_type=jnp.float32,
                                precision=_PREC)
        out_ref[0] = jax.nn.sigmoid(o + linb_ref[...])


def kernel(x, edge_index, Wz, bz, Wr, br, Wh, bh, lzW, lzb, lrW, lrb,
           lhW, lhb, att, linW, linb):
    f32 = jnp.float32
    loop = jnp.arange(_N, dtype=edge_index.dtype)
    row = jnp.concatenate([edge_index[0], loop])
    col = jnp.concatenate([edge_index[1], loop])
    pad_e = _EP - row.shape[0]
    row = jnp.pad(row, (0, pad_e), constant_values=_N).astype(jnp.int32)
    col = jnp.pad(col, (0, pad_e), constant_values=_N).astype(jnp.int32)
    col2 = col.reshape(1, _EP)

    xnm = jnp.pad(x.transpose(1, 0, 3, 2).reshape(_N, _C),
                  ((0, _NP - _N), (0, _CP - _C)))
    xnm2 = jnp.concatenate([xnm[:, :_CH], xnm[:, _CH:]], axis=0)

    lz = jnp.pad(lzW[:, :_MID], ((0, _MP - _MID), (0, _MP - _MID)))
    lh = jnp.pad(lhW[:, :_MID], ((0, _MP - _MID), (0, _MP - _MID)))
    wz = jnp.pad(Wz, ((0, _MP - _MID), (0, 0)))
    wh = jnp.pad(Wh, ((0, _MP - _MID), (0, 0)))
    bz2 = jnp.pad(bz, (0, _MP - _MID)).reshape(1, _MP)
    bh2 = jnp.pad(bh, (0, _MP - _MID)).reshape(1, _MP)
    lzb2 = jnp.pad(lzb, (0, _MP - _MID)).reshape(1, _MP)
    lhb2 = jnp.pad(lhb, (0, _MP - _MID)).reshape(1, _MP)
    lin = jnp.pad(linW, ((0, 0), (0, _MP - _MID)))
    linb2 = linb.reshape(1, _P)
    att2 = att.reshape(1, _P)

    xs, dis, cz, ch, dz, dh = pl.pallas_call(
        _prep_body,
        out_shape=(
            jax.ShapeDtypeStruct((2 * _NP, _CH), f32),
            jax.ShapeDtypeStruct((_NP, 1), f32),
            jax.ShapeDtypeStruct((_F, _MP), f32),
            jax.ShapeDtypeStruct((_F, _MP), f32),
            jax.ShapeDtypeStruct((1, _MP), f32),
            jax.ShapeDtypeStruct((1, _MP), f32),
        ),
    )(col2, xnm2, wz, wh, lz, lh, bz2, lzb2, bh2, lhb2)

    zero = jnp.zeros((_NP, _CH), f32)
    mesh = plsc.VectorSubcoreMesh(core_axis_name="c", subcore_axis_name="s")
    sc_agg = functools.partial(
        pl.kernel, mesh=mesh,
        out_type=jax.ShapeDtypeStruct((32, _NP, _CH), f32),
        scratch_types=[
            pltpu.VMEM((_ET,), jnp.int32),
            pltpu.VMEM((_ET,), jnp.int32),
            pltpu.VMEM((_ET, _CH), f32),
            pltpu.VMEM((_NP, _CH), f32),
            pltpu.SemaphoreType.DMA,
        ],
        compiler_params=pltpu.CompilerParams(needs_layout_passes=False),
    )(_sc_agg_body)
    ypart = sc_agg(row, col, xs, zero)

    ysum = pl.pallas_call(
        _reduce_body,
        grid=(2,),
        in_specs=[pl.BlockSpec((1, 16, _NP, _CH), lambda h: (h, 0, 0, 0))],
        out_specs=pl.BlockSpec((1, _NP, _CH), lambda h: (h, 0, 0)),
        out_shape=jax.ShapeDtypeStruct((2, _NP, _CH), f32),
    )(ypart.reshape(2, 16, _NP, _CH))

    yfull = jnp.concatenate([ysum[0], ysum[1]], axis=1)[:, :_C]
    yt = yfull.reshape(_NP, _BP, _F).transpose(1, 0, 2)
    y0 = yt[:, :, 0:1]
    y1 = yt[:, :, 1:2]

    full = lambda s: pl.BlockSpec(s, lambda b, p: tuple(0 for _ in s))
    bp_spec = pl.BlockSpec((1, _NP, 1), lambda b, p: (b * _P + p, 0, 0))
    out3 = pl.pallas_call(
        _gate_body,
        grid=(_B, _P),
        in_specs=[
            bp_spec, bp_spec,
            full((_NP, 1)),
            full((_F, _MP)),
            full((_F, _MP)),
            full((1, _MP)),
            full((1, _MP)),
            full((1, _P)),
            full((_P, _MP)),
            full((1, _P)),
        ],
        out_specs=pl.BlockSpec((1, _NP, _P), lambda b, p: (b, 0, 0)),
        out_shape=jax.ShapeDtypeStruct((_B, _NP, _P), f32),
        scratch_shapes=[pltpu.VMEM((_NP, _MP), f32)],
        compiler_params=pltpu.CompilerParams(
            dimension_semantics=("arbitrary", "arbitrary")),
    )(y0, y1, dis, cz, ch, dz, dh, att2, lin, linb2)

    return out3[:, :_N, :]


# speedup vs baseline: 1.4428x; 1.4428x over previous
import functools

import jax
import jax.numpy as jnp
from jax import lax
from jax.experimental import pallas as pl
from jax.experimental.pallas import tpu as pltpu
from jax.experimental.pallas import tpu_sc as plsc

_N = 207
_F = 2
_P = 12
_B = 12
_MID = _N * 5
_E = 1722
_BP = _B * _P
_C = _BP * _F
_CP = 512
_CH = _CP // 2

_NP = 208
_EP = 2048
_MP = 1152

_ET = _EP // 16

_PREC = jax.lax.Precision.HIGHEST


def _prep_body(col_ref, xnm_ref, wz_ref, wh_ref, lz_ref, lh_ref,
               bz_ref, lzb_ref, bh_ref, lhb_ref,
               xs_ref, dis_ref, cz_ref, ch_ref, dz_ref, dh_ref):
    col = col_ref[...]
    ids = jax.lax.broadcasted_iota(jnp.int32, (_NP, _EP), 0)
    oh_col = (ids == col).astype(jnp.float32)
    deg = jnp.sum(oh_col, axis=1, keepdims=True)
    dis = jnp.where(deg > 0.0, jax.lax.rsqrt(deg), 0.0)
    dis_ref[...] = dis
    xs_ref[0:_NP, :] = xnm_ref[0:_NP, :] * dis
    xs_ref[_NP:, :] = xnm_ref[_NP:, :] * dis
    cz_ref[...] = jax.lax.dot_general(wz_ref[...], lz_ref[...],
                                      (((0,), (1,)), ((), ())),
                                      preferred_element_type=jnp.float32,
                                      precision=_PREC)
    ch_ref[...] = jax.lax.dot_general(wh_ref[...], lh_ref[...],
                                      (((0,), (1,)), ((), ())),
                                      preferred_element_type=jnp.float32,
                                      precision=_PREC)
    dz_ref[...] = jax.lax.dot_general(bz_ref[...], lz_ref[...],
                                      (((1,), (1,)), ((), ())),
                                      preferred_element_type=jnp.float32,
                                      precision=_PREC) + lzb_ref[...]
    dh_ref[...] = jax.lax.dot_general(bh_ref[...], lh_ref[...],
                                      (((1,), (1,)), ((), ())),
                                      preferred_element_type=jnp.float32,
                                      precision=_PREC) + lhb_ref[...]


def _sc_agg_body(row_hbm, col_hbm, xs_hbm, zero_hbm, y_hbm,
                 row_v, col_v, rows_v, acc, sem):
    c = lax.axis_index("c")
    s = lax.axis_index("s")
    base = s * _ET

    pltpu.sync_copy(zero_hbm, acc)
    pltpu.sync_copy(row_hbm.at[pl.ds(base, _ET)], row_v)
    pltpu.sync_copy(col_hbm.at[pl.ds(base, _ET)], col_v)

    def _adj(i, _):
        row_v[pl.ds(i * 16, 16)] = row_v[pl.ds(i * 16, 16)] + c * _NP
        return 0
    lax.fori_loop(0, _ET // 16, _adj, 0)

    pltpu.async_copy(xs_hbm.at[row_v], rows_v, sem).wait()

    lanes = jax.lax.broadcasted_iota(jnp.int32, (16,), 0)

    def _edge_group(g, _):
        for k in range(16):
            eidx = jnp.full((16,), g * 16 + k, jnp.int32)
            ce = plsc.load_gather(col_v, [eidx])
            for j in range(_CH // 16):
                cidx = lanes + (j * 16)
                vals = plsc.load_gather(rows_v, [eidx, cidx])
                plsc.addupdate_scatter(acc, [ce, cidx], vals)
        return 0
    lax.fori_loop(0, _ET // 16, _edge_group, 0)

    pltpu.sync_copy(acc, y_hbm.at[c * 16 + s])


def _reduce_body(yp_ref, out_ref):
    out_ref[0] = jnp.sum(yp_ref[0], axis=0)


def _gate_body(y0_ref, y1_ref, dis_ref, cz_ref, ch_ref,
               dz_ref, dh_ref, att_ref, lin_ref, linb_ref, out_ref):
    dis = dis_ref[...]
    att = att_ref[...]
    e = jnp.exp(att - jnp.max(att))
    w_all = e / jnp.sum(e)
    cz0 = cz_ref[0:1, :_MID]
    cz1 = cz_ref[1:2, :_MID]
    ch0 = ch_ref[0:1, :_MID]
    ch1 = ch_ref[1:2, :_MID]
    dzv = dz_ref[0:1, :_MID]
    dhv = dh_ref[0:1, :_MID]
    acc = jnp.zeros((_NP, _MID), jnp.float32)
    for p in range(_P):
        y0 = y0_ref[0, p] * dis
        y1 = y1_ref[0, p] * dis
        pz = y0 * cz0 + y1 * cz1 + dzv
        ph = y0 * ch0 + y1 * ch1 + dhv
        hp = (1.0 - jax.nn.sigmoid(pz)) * jnp.tanh(ph)
        acc = acc + w_all[0:1, p:p + 1] * hp
    h = jnp.maximum(acc, 0.0)
    o = jax.lax.dot_general(h, lin_ref[:, :_MID],
                            (((1,), (1,)), ((), ())),
                            preferred_element_type=jnp.float32,
                            precision=_PREC)
    out_ref[0] = jax.nn.sigmoid(o + linb_ref[...])


def kernel(x, edge_index, Wz, bz, Wr, br, Wh, bh, lzW, lzb, lrW, lrb,
           lhW, lhb, att, linW, linb):
    f32 = jnp.float32
    loop = jnp.arange(_N, dtype=edge_index.dtype)
    row = jnp.concatenate([edge_index[0], loop])
    col = jnp.concatenate([edge_index[1], loop])
    pad_e = _EP - row.shape[0]
    row = jnp.pad(row, (0, pad_e), constant_values=_N).astype(jnp.int32)
    col = jnp.pad(col, (0, pad_e), constant_values=_N).astype(jnp.int32)
    col2 = col.reshape(1, _EP)

    xnm = jnp.pad(x.transpose(1, 0, 3, 2).reshape(_N, _C),
                  ((0, _NP - _N), (0, _CP - _C)))
    xnm2 = jnp.concatenate([xnm[:, :_CH], xnm[:, _CH:]], axis=0)

    lz = jnp.pad(lzW[:, :_MID], ((0, _MP - _MID), (0, _MP - _MID)))
    lh = jnp.pad(lhW[:, :_MID], ((0, _MP - _MID), (0, _MP - _MID)))
    wz = jnp.pad(Wz, ((0, _MP - _MID), (0, 0)))
    wh = jnp.pad(Wh, ((0, _MP - _MID), (0, 0)))
    bz2 = jnp.pad(bz, (0, _MP - _MID)).reshape(1, _MP)
    bh2 = jnp.pad(bh, (0, _MP - _MID)).reshape(1, _MP)
    lzb2 = jnp.pad(lzb, (0, _MP - _MID)).reshape(1, _MP)
    lhb2 = jnp.pad(lhb, (0, _MP - _MID)).reshape(1, _MP)
    lin = jnp.pad(linW, ((0, 0), (0, _MP - _MID)))
    linb2 = linb.reshape(1, _P)
    att2 = att.reshape(1, _P)

    xs, dis, cz, ch, dz, dh = pl.pallas_call(
        _prep_body,
        out_shape=(
            jax.ShapeDtypeStruct((2 * _NP, _CH), f32),
            jax.ShapeDtypeStruct((_NP, 1), f32),
            jax.ShapeDtypeStruct((_F, _MP), f32),
            jax.ShapeDtypeStruct((_F, _MP), f32),
            jax.ShapeDtypeStruct((1, _MP), f32),
            jax.ShapeDtypeStruct((1, _MP), f32),
        ),
    )(col2, xnm2, wz, wh, lz, lh, bz2, lzb2, bh2, lhb2)

    zero = jnp.zeros((_NP, _CH), f32)
    mesh = plsc.VectorSubcoreMesh(core_axis_name="c", subcore_axis_name="s")
    sc_agg = functools.partial(
        pl.kernel, mesh=mesh,
        out_type=jax.ShapeDtypeStruct((32, _NP, _CH), f32),
        scratch_types=[
            pltpu.VMEM((_ET,), jnp.int32),
            pltpu.VMEM((_ET,), jnp.int32),
            pltpu.VMEM((_ET, _CH), f32),
            pltpu.VMEM((_NP, _CH), f32),
            pltpu.SemaphoreType.DMA,
        ],
        compiler_params=pltpu.CompilerParams(needs_layout_passes=False),
    )(_sc_agg_body)
    ypart = sc_agg(row, col, xs, zero)

    ysum = pl.pallas_call(
        _reduce_body,
        grid=(2,),
        in_specs=[pl.BlockSpec((1, 16, _NP, _CH), lambda h: (h, 0, 0, 0))],
        out_specs=pl.BlockSpec((1, _NP, _CH), lambda h: (h, 0, 0)),
        out_shape=jax.ShapeDtypeStruct((2, _NP, _CH), f32),
    )(ypart.reshape(2, 16, _NP, _CH))

    yfull = jnp.concatenate([ysum[0], ysum[1]], axis=1)[:, :_C]
    yt = yfull.reshape(_NP, _BP, _F).transpose(1, 0, 2)
    y0 = yt[:, :, 0:1].reshape(_B, _P, _NP, 1)
    y1 = yt[:, :, 1:2].reshape(_B, _P, _NP, 1)

    full = lambda s: pl.BlockSpec(s, lambda b: tuple(0 for _ in s))
    b_spec = pl.BlockSpec((1, _P, _NP, 1), lambda b: (b, 0, 0, 0))
    out3 = pl.pallas_call(
        _gate_body,
        grid=(_B,),
        in_specs=[
            b_spec, b_spec,
            full((_NP, 1)),
            full((_F, _MP)),
            full((_F, _MP)),
            full((1, _MP)),
            full((1, _MP)),
            full((1, _P)),
            full((_P, _MP)),
            full((1, _P)),
        ],
        out_specs=pl.BlockSpec((1, _NP, _P), lambda b: (b, 0, 0)),
        out_shape=jax.ShapeDtypeStruct((_B, _NP, _P), f32),
        compiler_params=pltpu.CompilerParams(
            dimension_semantics=("arbitrary",)),
    )(y0, y1, dis, cz, ch, dz, dh, att2, lin, linb2)

    return out3[:, :_N, :]


# speedup vs baseline: 1.4527x; 1.0069x over previous
import functools

import jax
import jax.numpy as jnp
from jax import lax
from jax.experimental import pallas as pl
from jax.experimental.pallas import tpu as pltpu
from jax.experimental.pallas import tpu_sc as plsc

_N = 207
_F = 2
_P = 12
_B = 12
_MID = _N * 5
_E = 1722
_BP = _B * _P
_C = _BP * _F
_CP = 512
_CH = _CP // 2

_NP = 208
_EP = 2048
_ET = _EP // 16

_PREC = jax.lax.Precision.HIGHEST


def _prep_body(col_ref, xnm_ref, wz_ref, wh_ref, lz_ref, lh_ref,
               bz_ref, lzb_ref, bh_ref, lhb_ref,
               xs_ref, dis_ref, cz_ref, ch_ref, dz_ref, dh_ref):
    col = col_ref[...]
    ids = jax.lax.broadcasted_iota(jnp.int32, (_NP, _EP), 0)
    oh_col = (ids == col).astype(jnp.float32)
    deg = jnp.sum(oh_col, axis=1, keepdims=True)
    dis = jnp.where(deg > 0.0, jax.lax.rsqrt(deg), 0.0)
    dis_ref[...] = dis
    xs_ref[0:_NP, :] = xnm_ref[0:_NP, :] * dis
    xs_ref[_NP:, :] = xnm_ref[_NP:, :] * dis
    cz_ref[...] = jax.lax.dot_general(wz_ref[...], lz_ref[...],
                                      (((0,), (1,)), ((), ())),
                                      preferred_element_type=jnp.float32,
                                      precision=_PREC)
    ch_ref[...] = jax.lax.dot_general(wh_ref[...], lh_ref[...],
                                      (((0,), (1,)), ((), ())),
                                      preferred_element_type=jnp.float32,
                                      precision=_PREC)
    dz_ref[...] = jax.lax.dot_general(bz_ref[...], lz_ref[...],
                                      (((1,), (1,)), ((), ())),
                                      preferred_element_type=jnp.float32,
                                      precision=_PREC) + lzb_ref[...]
    dh_ref[...] = jax.lax.dot_general(bh_ref[...], lh_ref[...],
                                      (((1,), (1,)), ((), ())),
                                      preferred_element_type=jnp.float32,
                                      precision=_PREC) + lhb_ref[...]


def _sc_agg_body(row_hbm, col_hbm, xs_hbm, zero_hbm, y_hbm,
                 row_v, col_v, rows_v, acc, sem):
    c = lax.axis_index("c")
    s = lax.axis_index("s")
    base = s * _ET

    pltpu.sync_copy(zero_hbm, acc)
    pltpu.sync_copy(row_hbm.at[pl.ds(base, _ET)], row_v)
    pltpu.sync_copy(col_hbm.at[pl.ds(base, _ET)], col_v)

    def _adj(i, _):
        row_v[pl.ds(i * 16, 16)] = row_v[pl.ds(i * 16, 16)] + c * _NP
        return 0
    lax.fori_loop(0, _ET // 16, _adj, 0)

    pltpu.async_copy(xs_hbm.at[row_v], rows_v, sem).wait()

    lanes = jax.lax.broadcasted_iota(jnp.int32, (16,), 0)

    def _edge_group(g, _):
        for k in range(16):
            eidx = jnp.full((16,), g * 16 + k, jnp.int32)
            ce = plsc.load_gather(col_v, [eidx])
            for j in range(_CH // 16):
                cidx = lanes + (j * 16)
                vals = plsc.load_gather(rows_v, [eidx, cidx])
                plsc.addupdate_scatter(acc, [ce, cidx], vals)
        return 0
    lax.fori_loop(0, _ET // 16, _edge_group, 0)

    pltpu.sync_copy(acc, y_hbm.at[c * 16 + s])


def _reduce_body(yp_ref, dis_ref, y0_ref, y1_ref):
    h0 = jnp.sum(yp_ref[0:16], axis=0)
    h1 = jnp.sum(yp_ref[16:32], axis=0)
    yfull = jnp.concatenate([h0, h1[:, :_C - _CH]], axis=1) * dis_ref[...]
    y0_ref[...] = jnp.transpose(yfull[:, :_BP], (1, 0))
    y1_ref[...] = jnp.transpose(yfull[:, _BP:], (1, 0))


def _gate_body(y0_ref, y1_ref, cz_ref, ch_ref,
               dz_ref, dh_ref, att_ref, lin_ref, linb_ref, out_ref):
    att = att_ref[...]
    e = jnp.exp(att - jnp.max(att))
    w_all = e / jnp.sum(e)
    cz0 = cz_ref[0:1, :_MID]
    cz1 = cz_ref[1:2, :_MID]
    ch0 = ch_ref[0:1, :_MID]
    ch1 = ch_ref[1:2, :_MID]
    dzv = dz_ref[0:1, :_MID]
    dhv = dh_ref[0:1, :_MID]
    acc = jnp.zeros((_NP, _MID), jnp.float32)
    for p in range(_P):
        y0 = y0_ref[0, p]
        y1 = y1_ref[0, p]
        pz = y0 * cz0 + y1 * cz1 + dzv
        ph = y0 * ch0 + y1 * ch1 + dhv
        hp = (1.0 - jax.nn.sigmoid(pz)) * jnp.tanh(ph)
        acc = acc + w_all[0:1, p:p + 1] * hp
    h = jnp.maximum(acc, 0.0)
    o = jax.lax.dot_general(h, lin_ref[:, :_MID],
                            (((1,), (1,)), ((), ())),
                            preferred_element_type=jnp.float32,
                            precision=_PREC)
    out_ref[0] = jax.nn.sigmoid(o + linb_ref[...])


def kernel(x, edge_index, Wz, bz, Wr, br, Wh, bh, lzW, lzb, lrW, lrb,
           lhW, lhb, att, linW, linb):
    f32 = jnp.float32
    loop = jnp.arange(_N, dtype=edge_index.dtype)
    row = jnp.concatenate([edge_index[0], loop])
    col = jnp.concatenate([edge_index[1], loop])
    pad_e = _EP - row.shape[0]
    row = jnp.pad(row, (0, pad_e), constant_values=_N).astype(jnp.int32)
    col = jnp.pad(col, (0, pad_e), constant_values=_N).astype(jnp.int32)
    col2 = col.reshape(1, _EP)

    xnm = jnp.pad(x.transpose(1, 2, 0, 3).reshape(_N, _C),
                  ((0, _NP - _N), (0, _CP - _C)))
    xnm2 = jnp.concatenate([xnm[:, :_CH], xnm[:, _CH:]], axis=0)

    lz = lzW[:, :_MID]
    lh = lhW[:, :_MID]
    wz = Wz
    wh = Wh
    bz2 = bz.reshape(1, _MID)
    bh2 = bh.reshape(1, _MID)
    lzb2 = lzb.reshape(1, _MID)
    lhb2 = lhb.reshape(1, _MID)
    lin = linW
    linb2 = linb.reshape(1, _P)
    att2 = att.reshape(1, _P)

    xs, dis, cz, ch, dz, dh = pl.pallas_call(
        _prep_body,
        out_shape=(
            jax.ShapeDtypeStruct((2 * _NP, _CH), f32),
            jax.ShapeDtypeStruct((_NP, 1), f32),
            jax.ShapeDtypeStruct((_F, _MID), f32),
            jax.ShapeDtypeStruct((_F, _MID), f32),
            jax.ShapeDtypeStruct((1, _MID), f32),
            jax.ShapeDtypeStruct((1, _MID), f32),
        ),
    )(col2, xnm2, wz, wh, lz, lh, bz2, lzb2, bh2, lhb2)

    zero = jnp.zeros((_NP, _CH), f32)
    mesh = plsc.VectorSubcoreMesh(core_axis_name="c", subcore_axis_name="s")
    sc_agg = functools.partial(
        pl.kernel, mesh=mesh,
        out_type=jax.ShapeDtypeStruct((32, _NP, _CH), f32),
        scratch_types=[
            pltpu.VMEM((_ET,), jnp.int32),
            pltpu.VMEM((_ET,), jnp.int32),
            pltpu.VMEM((_ET, _CH), f32),
            pltpu.VMEM((_NP, _CH), f32),
            pltpu.SemaphoreType.DMA,
        ],
        compiler_params=pltpu.CompilerParams(needs_layout_passes=False),
    )(_sc_agg_body)
    ypart = sc_agg(row, col, xs, zero)

    y0t, y1t = pl.pallas_call(
        _reduce_body,
        out_shape=(
            jax.ShapeDtypeStruct((_BP, _NP), f32),
            jax.ShapeDtypeStruct((_BP, _NP), f32),
        ),
    )(ypart, dis)
    y0 = y0t.reshape(_B, _P, _NP, 1)
    y1 = y1t.reshape(_B, _P, _NP, 1)

    full = lambda s: pl.BlockSpec(s, lambda b: tuple(0 for _ in s))
    b_spec = pl.BlockSpec((1, _P, _NP, 1), lambda b: (b, 0, 0, 0))
    out3 = pl.pallas_call(
        _gate_body,
        grid=(_B,),
        in_specs=[
            b_spec, b_spec,
            full((_F, _MID)),
            full((_F, _MID)),
            full((1, _MID)),
            full((1, _MID)),
            full((1, _P)),
            full((_P, _MID)),
            full((1, _P)),
        ],
        out_specs=pl.BlockSpec((1, _NP, _P), lambda b: (b, 0, 0)),
        out_shape=jax.ShapeDtypeStruct((_B, _NP, _P), f32),
        compiler_params=pltpu.CompilerParams(
            dimension_semantics=("arbitrary",)),
    )(y0, y1, cz, ch, dz, dh, att2, lin, linb2)

    return out3[:, :_N, :]


# speedup vs baseline: 1.5922x; 1.0960x over previous
import functools

import jax
import jax.numpy as jnp
from jax import lax
from jax.experimental import pallas as pl
from jax.experimental.pallas import tpu as pltpu
from jax.experimental.pallas import tpu_sc as plsc

_N = 207
_F = 2
_P = 12
_B = 12
_MID = _N * 5
_E = 1722
_BP = _B * _P
_C = _BP * _F
_CP = 512
_CH = _CP // 2

_NP = 208
_EP = 2048
_ET = _EP // 16

_PREC = jax.lax.Precision.HIGHEST


def _graph_prep_body(col_ref, xnm_ref, xs_ref, dis_ref):
    col = col_ref[...]
    ids = jax.lax.broadcasted_iota(jnp.int32, (_NP, _EP), 0)
    oh_col = (ids == col).astype(jnp.float32)
    deg = jnp.sum(oh_col, axis=1, keepdims=True)
    dis = jnp.where(deg > 0.0, jax.lax.rsqrt(deg), 0.0)
    dis_ref[...] = dis
    xs_ref[0:_NP, :] = xnm_ref[0:_NP, :] * dis
    xs_ref[_NP:, :] = xnm_ref[_NP:, :] * dis


def _fold_body(wz_ref, wh_ref, lz_ref, lh_ref,
               bz_ref, lzb_ref, bh_ref, lhb_ref,
               cz_ref, ch_ref, dz_ref, dh_ref):
    cz_ref[...] = jax.lax.dot_general(wz_ref[...], lz_ref[...],
                                      (((0,), (1,)), ((), ())),
                                      preferred_element_type=jnp.float32,
                                      precision=_PREC)
    ch_ref[...] = jax.lax.dot_general(wh_ref[...], lh_ref[...],
                                      (((0,), (1,)), ((), ())),
                                      preferred_element_type=jnp.float32,
                                      precision=_PREC)
    dz_ref[...] = jax.lax.dot_general(bz_ref[...], lz_ref[...],
                                      (((1,), (1,)), ((), ())),
                                      preferred_element_type=jnp.float32,
                                      precision=_PREC) + lzb_ref[...]
    dh_ref[...] = jax.lax.dot_general(bh_ref[...], lh_ref[...],
                                      (((1,), (1,)), ((), ())),
                                      preferred_element_type=jnp.float32,
                                      precision=_PREC) + lhb_ref[...]


def _sc_agg_body(row_hbm, col_hbm, xs_hbm, zero_hbm, y_hbm,
                 row_v, col_v, rows_v, acc, sem):
    c = lax.axis_index("c")
    s = lax.axis_index("s")
    base = s * _ET

    pltpu.sync_copy(zero_hbm, acc)
    pltpu.sync_copy(row_hbm.at[pl.ds(base, _ET)], row_v)
    pltpu.sync_copy(col_hbm.at[pl.ds(base, _ET)], col_v)

    def _adj(i, _):
        row_v[pl.ds(i * 16, 16)] = row_v[pl.ds(i * 16, 16)] + c * _NP
        return 0
    lax.fori_loop(0, _ET // 16, _adj, 0)

    pltpu.async_copy(xs_hbm.at[row_v], rows_v, sem).wait()

    lanes = jax.lax.broadcasted_iota(jnp.int32, (16,), 0)

    def _edge_group(g, _):
        for k in range(16):
            eidx = jnp.full((16,), g * 16 + k, jnp.int32)
            ce = plsc.load_gather(col_v, [eidx])
            for j in range(_CH // 16):
                cidx = lanes + (j * 16)
                vals = plsc.load_gather(rows_v, [eidx, cidx])
                plsc.addupdate_scatter(acc, [ce, cidx], vals)
        return 0
    lax.fori_loop(0, _ET // 16, _edge_group, 0)

    pltpu.sync_copy(acc, y_hbm.at[c * 16 + s])


def _reduce_body(yp_ref, dis_ref, y0_ref, y1_ref):
    h0 = jnp.sum(yp_ref[0:16], axis=0)
    h1 = jnp.sum(yp_ref[16:32], axis=0)
    yfull = jnp.concatenate([h0, h1[:, :_C - _CH]], axis=1) * dis_ref[...]
    y0_ref[...] = jnp.transpose(yfull[:, :_BP], (1, 0))
    y1_ref[...] = jnp.transpose(yfull[:, _BP:], (1, 0))


def _gate_body(y0_ref, y1_ref, cz_ref, ch_ref,
               dz_ref, dh_ref, att_ref, lin_ref, linb_ref, out_ref):
    att = att_ref[...]
    e = jnp.exp(att - jnp.max(att))
    w_all = e / jnp.sum(e)
    cz0 = cz_ref[0:1, :_MID]
    cz1 = cz_ref[1:2, :_MID]
    ch0 = ch_ref[0:1, :_MID]
    ch1 = ch_ref[1:2, :_MID]
    dzv = dz_ref[0:1, :_MID]
    dhv = dh_ref[0:1, :_MID]
    acc = jnp.zeros((_NP, _MID), jnp.float32)
    for p in range(_P):
        y0 = y0_ref[0, p]
        y1 = y1_ref[0, p]
        pz = y0 * cz0 + y1 * cz1 + dzv
        ph = y0 * ch0 + y1 * ch1 + dhv
        hp = (1.0 - jax.nn.sigmoid(pz)) * jnp.tanh(ph)
        acc = acc + w_all[0:1, p:p + 1] * hp
    h = jnp.maximum(acc, 0.0)
    o = jax.lax.dot_general(h, lin_ref[:, :_MID],
                            (((1,), (1,)), ((), ())),
                            preferred_element_type=jnp.float32,
                            precision=_PREC)
    out_ref[0] = jax.nn.sigmoid(o + linb_ref[...])


def kernel(x, edge_index, Wz, bz, Wr, br, Wh, bh, lzW, lzb, lrW, lrb,
           lhW, lhb, att, linW, linb):
    f32 = jnp.float32
    loop = jnp.arange(_N, dtype=edge_index.dtype)
    row = jnp.concatenate([edge_index[0], loop])
    col = jnp.concatenate([edge_index[1], loop])
    pad_e = _EP - row.shape[0]
    row = jnp.pad(row, (0, pad_e), constant_values=_N).astype(jnp.int32)
    col = jnp.pad(col, (0, pad_e), constant_values=_N).astype(jnp.int32)
    col2 = col.reshape(1, _EP)

    xnm = jnp.pad(x.transpose(1, 2, 0, 3).reshape(_N, _C),
                  ((0, _NP - _N), (0, _CP - _C)))
    xnm2 = jnp.concatenate([xnm[:, :_CH], xnm[:, _CH:]], axis=0)

    lz = lzW[:, :_MID]
    lh = lhW[:, :_MID]
    wz = Wz
    wh = Wh
    bz2 = bz.reshape(1, _MID)
    bh2 = bh.reshape(1, _MID)
    lzb2 = lzb.reshape(1, _MID)
    lhb2 = lhb.reshape(1, _MID)
    lin = linW
    linb2 = linb.reshape(1, _P)
    att2 = att.reshape(1, _P)

    xs, dis = pl.pallas_call(
        _graph_prep_body,
        out_shape=(
            jax.ShapeDtypeStruct((2 * _NP, _CH), f32),
            jax.ShapeDtypeStruct((_NP, 1), f32),
        ),
    )(col2, xnm2)

    cz, ch, dz, dh = pl.pallas_call(
        _fold_body,
        out_shape=(
            jax.ShapeDtypeStruct((_F, _MID), f32),
            jax.ShapeDtypeStruct((_F, _MID), f32),
            jax.ShapeDtypeStruct((1, _MID), f32),
            jax.ShapeDtypeStruct((1, _MID), f32),
        ),
    )(wz, wh, lz, lh, bz2, lzb2, bh2, lhb2)

    zero = jnp.zeros((_NP, _CH), f32)
    mesh = plsc.VectorSubcoreMesh(core_axis_name="c", subcore_axis_name="s")
    sc_agg = functools.partial(
        pl.kernel, mesh=mesh,
        out_type=jax.ShapeDtypeStruct((32, _NP, _CH), f32),
        scratch_types=[
            pltpu.VMEM((_ET,), jnp.int32),
            pltpu.VMEM((_ET,), jnp.int32),
            pltpu.VMEM((_ET, _CH), f32),
            pltpu.VMEM((_NP, _CH), f32),
            pltpu.SemaphoreType.DMA,
        ],
        compiler_params=pltpu.CompilerParams(needs_layout_passes=False),
    )(_sc_agg_body)
    ypart = sc_agg(row, col, xs, zero)

    y0t, y1t = pl.pallas_call(
        _reduce_body,
        out_shape=(
            jax.ShapeDtypeStruct((_BP, _NP), f32),
            jax.ShapeDtypeStruct((_BP, _NP), f32),
        ),
    )(ypart, dis)
    y0 = y0t.reshape(_B, _P, _NP, 1)
    y1 = y1t.reshape(_B, _P, _NP, 1)

    full = lambda s: pl.BlockSpec(s, lambda b: tuple(0 for _ in s))
    b_spec = pl.BlockSpec((1, _P, _NP, 1), lambda b: (b, 0, 0, 0))
    out3 = pl.pallas_call(
        _gate_body,
        grid=(_B,),
        in_specs=[
            b_spec, b_spec,
            full((_F, _MID)),
            full((_F, _MID)),
            full((1, _MID)),
            full((1, _MID)),
            full((1, _P)),
            full((_P, _MID)),
            full((1, _P)),
        ],
        out_specs=pl.BlockSpec((1, _NP, _P), lambda b: (b, 0, 0)),
        out_shape=jax.ShapeDtypeStruct((_B, _NP, _P), f32),
        compiler_params=pltpu.CompilerParams(
            dimension_semantics=("arbitrary",)),
    )(y0, y1, cz, ch, dz, dh, att2, lin, linb2)

    return out3[:, :_N, :]
